# Initial kernel scaffold; baseline (speedup 1.0000x reference)
import jax
import jax.numpy as jnp
from jax.experimental import pallas as pl


def kernel(input, weights):
    raise NotImplementedError("write your pallas kernel here")


# speedup vs baseline: 2.0109x; 2.0109x over previous
import functools

import jax
import jax.numpy as jnp
from jax import lax
from jax.experimental import pallas as pl
from jax.experimental.pallas import tpu as pltpu
from jax.experimental.pallas import tpu_sc as plsc

_NC = 2
_NS = 16
_L = 16
_NW = _NC * _NS


def _lane_consts():
  lane = lax.broadcasted_iota(jnp.int32, (_L,), 0)
  zeros = jnp.full((_L,), 0, jnp.int32)
  ones = jnp.full((_L,), 1, jnp.int32)
  return lane, zeros, ones


def _take(x, idx):
  return x.at[idx].get(mode="promise_in_bounds")


def _lane_cumsum(x, lane, zeros, ones):
  for k in (1, 2, 4, 8):
    shifted = _take(x, jnp.maximum(lane - k, zeros))
    gate = jnp.minimum(jnp.maximum(lane - (k - 1), zeros), ones)
    x = x + shifted * gate
  return x


def _bcast_last(x):
  return _take(x, jnp.full((_L,), _L - 1, jnp.int32))


def _make_sc_embed(bsz, seq_len, vocab_rows, dim):
  B = bsz * seq_len
  rpw = B // _NW
  wpr = _NW // bsz
  chunk = 32
  nch = rpw // chunk
  nvec = rpw // _L

  mesh = plsc.VectorSubcoreMesh(core_axis_name="c", subcore_axis_name="s")

  @functools.partial(
      pl.kernel,
      mesh=mesh,
      out_type=jax.ShapeDtypeStruct((B, dim), jnp.float32),
      scratch_types=[
          pltpu.VMEM((seq_len,), jnp.int32),
          pltpu.VMEM((rpw,), jnp.int32),
          pltpu.VMEM((chunk, dim), jnp.float32),
          pltpu.VMEM((chunk, dim), jnp.float32),
          pltpu.SemaphoreType.DMA,
          pltpu.SemaphoreType.DMA,
          pltpu.SemaphoreType.DMA,
          pltpu.SemaphoreType.DMA,
      ],
  )
  def sc_embed(in_hbm, w_hbm, out_hbm, tok_v, idx_v, buf0, buf1,
               sg0, sg1, sw0, sw1):
    wid = lax.axis_index("s") * _NC + lax.axis_index("c")
    b = wid // wpr
    j = wid - b * wpr

    pltpu.sync_copy(in_hbm.at[pl.ds(b * seq_len, seq_len)], tok_v)

    lane, zeros, ones = _lane_consts()

    def pref_body(i, acc):
      t = tok_v[pl.ds(i * _L, _L)]
      return acc + jnp.minimum(t, ones)

    acc = lax.fori_loop(0, j * nvec, pref_body, zeros)
    prefix = _bcast_last(_lane_cumsum(acc, lane, zeros, ones))

    s0 = j * rpw

    def pos_body(i, carry):
      t = tok_v[pl.ds(s0 + i * _L, _L)]
      m = jnp.minimum(t, ones)
      c = _lane_cumsum(m, lane, zeros, ones) + carry
      pos = c * m
      idx_v[pl.ds(i * _L, _L)] = pos
      return _bcast_last(c)

    lax.fori_loop(0, nvec, pos_body, prefix)

    @pl.when(j == 0)
    def _():
      onehot0 = jnp.maximum(ones - lane, zeros)
      head = idx_v[pl.ds(0, _L)]
      idx_v[pl.ds(0, _L)] = head + (ones - head) * onehot0

    out_base = wid * rpw
    bufs = (buf0, buf1)
    gsems = (sg0, sg1)
    wsems = (sw0, sw1)

    def start_gather(c, bsel):
      return pltpu.async_copy(
          w_hbm.at[idx_v.at[pl.ds(c * chunk, chunk)]], bufs[bsel],
          gsems[bsel])

    def start_write(c, bsel):
      return pltpu.async_copy(
          bufs[bsel], out_hbm.at[pl.ds(out_base + c * chunk, chunk)],
          wsems[bsel])

    gathers = [None] * nch
    writes = [None] * nch
    gathers[0] = start_gather(0, 0)
    for c in range(nch):
      bsel = c % 2
      if c + 1 < nch:
        nb = (c + 1) % 2
        if c >= 1:
          writes[c - 1].wait()
        gathers[c + 1] = start_gather(c + 1, nb)
      gathers[c].wait()
      writes[c] = start_write(c, bsel)
    writes[nch - 2].wait()
    writes[nch - 1].wait()

  return sc_embed


def kernel(input, weights):
  bsz, seq_len = input.shape
  vocab_rows, dim = weights.shape
  flat_in = input.reshape(-1)
  sc_embed = _make_sc_embed(bsz, seq_len, vocab_rows, dim)
  out = sc_embed(flat_in, weights)
  return out.reshape(bsz, seq_len, dim)


# speedup vs baseline: 2.0222x; 1.0056x over previous
import functools

import jax
import jax.numpy as jnp
from jax import lax
from jax.experimental import pallas as pl
from jax.experimental.pallas import tpu as pltpu
from jax.experimental.pallas import tpu_sc as plsc

_NC = 2
_NS = 16
_L = 16
_NW = _NC * _NS


def _lane_consts():
  lane = lax.broadcasted_iota(jnp.int32, (_L,), 0)
  zeros = jnp.full((_L,), 0, jnp.int32)
  ones = jnp.full((_L,), 1, jnp.int32)
  return lane, zeros, ones


def _take(x, idx):
  return x.at[idx].get(mode="promise_in_bounds")


def _lane_cumsum(x, lane, zeros, ones):
  for k in (1, 2, 4, 8):
    shifted = _take(x, jnp.maximum(lane - k, zeros))
    gate = jnp.minimum(jnp.maximum(lane - (k - 1), zeros), ones)
    x = x + shifted * gate
  return x


def _bcast_last(x):
  return _take(x, jnp.full((_L,), _L - 1, jnp.int32))


def _make_sc_embed(bsz, seq_len, vocab_rows, dim):
  B = bsz * seq_len
  rpw = B // _NW
  wpr = _NW // bsz
  chunk = 32
  nch = rpw // chunk
  nvec = rpw // _L

  mesh = plsc.VectorSubcoreMesh(core_axis_name="c", subcore_axis_name="s")

  @functools.partial(
      pl.kernel,
      mesh=mesh,
      out_type=jax.ShapeDtypeStruct((B, dim), jnp.float32),
      scratch_types=[
          pltpu.VMEM((seq_len,), jnp.int32),
          pltpu.VMEM((rpw,), jnp.int32),
          pltpu.VMEM((chunk, dim), jnp.float32),
          pltpu.VMEM((chunk, dim), jnp.float32),
          pltpu.VMEM((chunk, dim), jnp.float32),
          pltpu.SemaphoreType.DMA,
          pltpu.SemaphoreType.DMA,
          pltpu.SemaphoreType.DMA,
          pltpu.SemaphoreType.DMA,
          pltpu.SemaphoreType.DMA,
          pltpu.SemaphoreType.DMA,
      ],
  )
  def sc_embed(in_hbm, w_hbm, out_hbm, tok_v, idx_v, buf0, buf1, buf2,
               sg0, sg1, sg2, sw0, sw1, sw2):
    wid = lax.axis_index("s") * _NC + lax.axis_index("c")
    b = wid // wpr
    j = wid - b * wpr

    pltpu.sync_copy(in_hbm.at[pl.ds(b * seq_len, seq_len)], tok_v)

    lane, zeros, ones = _lane_consts()

    def pref_body(i, acc):
      t = tok_v[pl.ds(i * _L, _L)]
      return acc + jnp.minimum(t, ones)

    acc = lax.fori_loop(0, j * nvec, pref_body, zeros)
    prefix = _bcast_last(_lane_cumsum(acc, lane, zeros, ones))

    s0 = j * rpw

    def pos_body(i, carry):
      t = tok_v[pl.ds(s0 + i * _L, _L)]
      m = jnp.minimum(t, ones)
      c = _lane_cumsum(m, lane, zeros, ones) + carry
      pos = c * m
      idx_v[pl.ds(i * _L, _L)] = pos
      return _bcast_last(c)

    lax.fori_loop(0, nvec, pos_body, prefix)

    @pl.when(j == 0)
    def _():
      onehot0 = jnp.maximum(ones - lane, zeros)
      head = idx_v[pl.ds(0, _L)]
      idx_v[pl.ds(0, _L)] = head + (ones - head) * onehot0

    out_base = wid * rpw
    nbuf = 3
    bufs = (buf0, buf1, buf2)
    gsems = (sg0, sg1, sg2)
    wsems = (sw0, sw1, sw2)

    def start_gather(c):
      return pltpu.async_copy(
          w_hbm.at[idx_v.at[pl.ds(c * chunk, chunk)]], bufs[c % nbuf],
          gsems[c % nbuf])

    def start_write(c):
      return pltpu.async_copy(
          bufs[c % nbuf], out_hbm.at[pl.ds(out_base + c * chunk, chunk)],
          wsems[c % nbuf])

    gathers = [None] * nch
    writes = [None] * nch
    gathers[0] = start_gather(0)
    if nch > 1:
      gathers[1] = start_gather(1)
    for c in range(nch):
      if c + 2 < nch:
        if c >= 1:
          writes[c - 1].wait()
        gathers[c + 2] = start_gather(c + 2)
      gathers[c].wait()
      writes[c] = start_write(c)
    for c in range(max(0, nch - nbuf), nch):
      writes[c].wait()

  return sc_embed


def kernel(input, weights):
  bsz, seq_len = input.shape
  vocab_rows, dim = weights.shape
  flat_in = input.reshape(-1)
  sc_embed = _make_sc_embed(bsz, seq_len, vocab_rows, dim)
  out = sc_embed(flat_in, weights)
  return out.reshape(bsz, seq_len, dim)
